# Initial kernel scaffold; baseline (speedup 1.0000x reference)
import jax
import jax.numpy as jnp
from jax.experimental import pallas as pl


def kernel(H, A, W, proj_W, proj_b):
    raise NotImplementedError("write your pallas kernel here")


# speedup vs baseline: 2.1612x; 2.1612x over previous
import functools

import jax
import jax.numpy as jnp
from jax import lax
from jax.experimental import pallas as pl
from jax.experimental.pallas import tpu as pltpu
from jax.experimental.pallas import tpu_sc as plsc

_N = 10000
_D = 128
_K = 5000
_NPAD = 16384
_SORT_R = 128
_SORT_C = 128
_KPAD = 5008
_NJ = _KPAD // 16
_RPAD = 5120

_NC = 2
_NS = 16
_NW = _NC * _NS
_CHUNK = 160
_TAIL = _K - (_NW - 1) * _CHUNK


def _roll(x, shift, axis):
    n = x.shape[axis]
    shift = shift % n
    if shift == 0:
        return x
    if axis == 0:
        return jnp.concatenate([x[shift:, :], x[:shift, :]], axis=0)
    return jnp.concatenate([x[:, shift:], x[:, :shift]], axis=1)


def _sort_body(s_ref, ks_ref, ki_ref):
    S = s_ref[...]
    row = lax.broadcasted_iota(jnp.int32, (_SORT_R, _SORT_C), 0)
    col = lax.broadcasted_iota(jnp.int32, (_SORT_R, _SORT_C), 1)
    L = row * _SORT_C + col
    I = L
    k = 2
    while k <= _NPAD:
        d = k // 2
        while d >= 1:
            if d < _SORT_C:
                ax, sh = 1, d
            else:
                ax, sh = 0, d // _SORT_C
            lower = (L & d) == 0
            oS = jnp.where(lower, _roll(S, sh, ax), _roll(S, -sh, ax))
            oI = jnp.where(lower, _roll(I, sh, ax), _roll(I, -sh, ax))
            lt = (S > oS) | ((S == oS) & (I < oI))
            asc = (L & k) == 0
            take = lt == (lower == asc)
            S = jnp.where(take, S, oS)
            I = jnp.where(take, I, oI)
            d //= 2
        k *= 2
    ks_ref[...] = S
    ki_ref[...] = I


def _topk_sort(spad):
    return pl.pallas_call(
        _sort_body,
        out_shape=(jax.ShapeDtypeStruct((_SORT_R, _SORT_C), jnp.float32),
                   jax.ShapeDtypeStruct((_SORT_R, _SORT_C), jnp.int32)),
    )(spad)


def _sc_pool(A, Hm, idx_c, idx_r1, idx_r2):
    mesh = plsc.VectorSubcoreMesh(core_axis_name="c", subcore_axis_name="s")

    @functools.partial(
        pl.kernel,
        out_type=(jax.ShapeDtypeStruct((_K * _K,), jnp.float32),
                  jax.ShapeDtypeStruct((_RPAD, _D), jnp.float32)),
        mesh=mesh,
        compiler_params=pltpu.CompilerParams(needs_layout_passes=False,
                                             use_tc_tiling_on_sc=False),
        scratch_types=[
            pltpu.VMEM((_KPAD,), jnp.int32),
            pltpu.VMEM((_CHUNK,), jnp.int32),
            pltpu.VMEM((_CHUNK // 2, 2), jnp.int32),
            pltpu.VMEM((2, 2, _N), jnp.float32),
            pltpu.VMEM((2 * 2 * _KPAD,), jnp.float32),
            pltpu.VMEM((32, _D), jnp.float32),
            pltpu.SemaphoreType.DMA((2,)),
            pltpu.SemaphoreType.DMA((2,)),
            pltpu.SemaphoreType.DMA,
        ],
    )
    def k(A_hbm, H_hbm, idxc_hbm, idxr1_hbm, idxr2_hbm, pA_hbm, pH_hbm,
          cidx, ridx1, ridx2, rowbuf, outbuf, hvbuf, sem_row, sem_out, sem_h):
        wid = lax.axis_index("s") * _NC + lax.axis_index("c")
        start = pl.multiple_of(wid * _CHUNK, _CHUNK)
        half = pl.multiple_of(wid * (_CHUNK // 2), _CHUNK // 2)
        nrows = jnp.where(wid < _NW - 1, _CHUNK, _TAIL)
        nt = nrows // 2

        pltpu.sync_copy(idxc_hbm, cidx)
        pltpu.sync_copy(idxr1_hbm.at[pl.ds(start, _CHUNK)], ridx1)
        pltpu.sync_copy(idxr2_hbm.at[pl.ds(half, _CHUNK // 2)], ridx2)

        for p in range(_CHUNK // 32):
            pltpu.async_copy(H_hbm.at[ridx1.at[pl.ds(32 * p, 32)]],
                             hvbuf, sem_h).wait()
            dst0 = pl.multiple_of(start + 32 * p, 8)
            pltpu.sync_copy(hvbuf, pH_hbm.at[pl.ds(dst0, 32)])

        def row_gather(t, tb):
            return pltpu.make_async_copy(A_hbm.at[ridx2.at[t]],
                                         rowbuf.at[tb], sem_row.at[tb])

        def out_store(t, tb, rr):
            src = pl.multiple_of((2 * tb + rr) * _KPAD, 16)
            dst = pl.multiple_of((start + 2 * t + rr) * _K, 8)
            return pltpu.make_async_copy(outbuf.at[pl.ds(src, _K)],
                                         pA_hbm.at[pl.ds(dst, _K)],
                                         sem_out.at[tb])

        row_gather(0, 0).start()

        def body(t, carry):
            tb = lax.rem(t, 2)

            @pl.when(t >= 2)
            def _():
                out_store(t - 2, tb, 0).wait()
                out_store(t - 2, tb, 1).wait()

            row_gather(t, tb).wait()

            @pl.when(t + 1 < nt)
            def _():
                row_gather(t + 1, 1 - tb).start()

            b0 = pl.multiple_of(2 * tb * _KPAD, 16)
            b1 = pl.multiple_of((2 * tb + 1) * _KPAD, 16)
            tbv = jnp.full((16,), tb, jnp.int32)
            r0v = jnp.zeros((16,), jnp.int32)
            r1v = jnp.ones((16,), jnp.int32)

            def cg(jv, c):
                cvec = cidx[pl.ds(jv * 16, 16)]
                outbuf[pl.ds(b0 + jv * 16, 16)] = plsc.load_gather(
                    rowbuf, [tbv, r0v, cvec])
                outbuf[pl.ds(b1 + jv * 16, 16)] = plsc.load_gather(
                    rowbuf, [tbv, r1v, cvec])
                return c

            lax.fori_loop(0, _NJ, cg, 0, unroll=4)
            out_store(t, tb, 0).start()
            out_store(t, tb, 1).start()
            return carry

        lax.fori_loop(0, nt, body, 0)
        out_store(nt - 2, lax.rem(nt - 2, 2), 0).wait()
        out_store(nt - 2, lax.rem(nt - 2, 2), 1).wait()
        out_store(nt - 1, lax.rem(nt - 1, 2), 0).wait()
        out_store(nt - 1, lax.rem(nt - 1, 2), 1).wait()

    return k(A, Hm, idx_c, idx_r1, idx_r2)


def _b1(pH, vals, W):
    def body(ph_ref, v_ref, w_ref, o_ref):
        o_ref[...] = jnp.dot(ph_ref[...] * v_ref[...], w_ref[...],
                             preferred_element_type=jnp.float32)

    return pl.pallas_call(
        body,
        out_shape=jax.ShapeDtypeStruct((_K, _D), jnp.float32),
    )(pH, vals, W)


def _b2(pA, HW):
    M = 1000

    def body(a_ref, hw_ref, o_ref):
        o_ref[...] = jnp.maximum(
            jnp.dot(a_ref[...], hw_ref[...],
                    preferred_element_type=jnp.float32), 0.0)

    return pl.pallas_call(
        body,
        grid=(_K // M,),
        in_specs=[pl.BlockSpec((M, _K), lambda i: (i, 0)),
                  pl.BlockSpec((_K, _D), lambda i: (0, 0))],
        out_specs=pl.BlockSpec((M, _D), lambda i: (i, 0)),
        out_shape=jax.ShapeDtypeStruct((_K, _D), jnp.float32),
    )(pA, HW)


def kernel(H, A, W, proj_W, proj_b):
    weights = (H @ proj_W + proj_b)[:, 0]
    scores = jax.nn.sigmoid(weights)

    spad = jnp.concatenate(
        [scores, jnp.full((_NPAD - _N,), -jnp.inf, jnp.float32)])
    ks, ki = _topk_sort(spad.reshape(_SORT_R, _SORT_C))
    values = ks.reshape(-1)[:_K]
    idx = ki.reshape(-1)[:_K]

    idx_c = jnp.concatenate([idx, jnp.zeros((_KPAD - _K,), jnp.int32)])
    idx_r = jnp.concatenate([idx, jnp.zeros((_RPAD - _K,), jnp.int32)])
    pAf, pHp = _sc_pool(A, H, idx_c, idx_r, idx_r.reshape(_RPAD // 2, 2))
    pA = pAf.reshape(_K, _K)
    pH = pHp[:_K]

    HW = _b1(pH, values.reshape(_K, 1), W)
    out = _b2(pA, HW)
    return (out, pA, idx)


# speedup vs baseline: 2.2724x; 1.0514x over previous
import functools

import jax
import jax.numpy as jnp
from jax import lax
from jax.experimental import pallas as pl
from jax.experimental.pallas import tpu as pltpu
from jax.experimental.pallas import tpu_sc as plsc

_N = 10000
_D = 128
_K = 5000
_NPAD = 16384
_SORT_R = 128
_SORT_C = 128
_KPAD = 5120
_NJ = _KPAD // 16
_NT = _KPAD // 128
_GRP = 8 * _KPAD
_RPAD = 5120

_NC = 2
_NS = 16
_NW = _NC * _NS
_CHUNK = 160
_TAIL = _K - (_NW - 1) * _CHUNK


def _roll(x, shift, axis):
    n = x.shape[axis]
    shift = shift % n
    if shift == 0:
        return x
    if axis == 0:
        return jnp.concatenate([x[shift:, :], x[:shift, :]], axis=0)
    return jnp.concatenate([x[:, shift:], x[:, :shift]], axis=1)


def _sort_body(s_ref, ks_ref, ki_ref):
    S = s_ref[...]
    row = lax.broadcasted_iota(jnp.int32, (_SORT_R, _SORT_C), 0)
    col = lax.broadcasted_iota(jnp.int32, (_SORT_R, _SORT_C), 1)
    L = row * _SORT_C + col
    I = L
    k = 2
    while k <= _NPAD:
        d = k // 2
        while d >= 1:
            if d < _SORT_C:
                ax, sh = 1, d
            else:
                ax, sh = 0, d // _SORT_C
            lower = (L & d) == 0
            oS = jnp.where(lower, _roll(S, sh, ax), _roll(S, -sh, ax))
            oI = jnp.where(lower, _roll(I, sh, ax), _roll(I, -sh, ax))
            lt = (S > oS) | ((S == oS) & (I < oI))
            asc = (L & k) == 0
            take = lt == (lower == asc)
            S = jnp.where(take, S, oS)
            I = jnp.where(take, I, oI)
            d //= 2
        k *= 2
    ks_ref[...] = S
    ki_ref[...] = I


def _topk_sort(spad):
    return pl.pallas_call(
        _sort_body,
        out_shape=(jax.ShapeDtypeStruct((_SORT_R, _SORT_C), jnp.float32),
                   jax.ShapeDtypeStruct((_SORT_R, _SORT_C), jnp.int32)),
    )(spad)


def _sc_pool(A, Hm, idx_pad, idx_pairs):
    mesh = plsc.VectorSubcoreMesh(core_axis_name="c", subcore_axis_name="s")

    @functools.partial(
        pl.kernel,
        out_type=(jax.ShapeDtypeStruct((_K // 8 * _GRP,), jnp.float32),
                  jax.ShapeDtypeStruct((_RPAD, _D), jnp.float32)),
        mesh=mesh,
        compiler_params=pltpu.CompilerParams(needs_layout_passes=False,
                                             use_tc_tiling_on_sc=False),
        scratch_types=[
            pltpu.VMEM((_KPAD,), jnp.int32),
            pltpu.VMEM((_CHUNK,), jnp.int32),
            pltpu.VMEM((_CHUNK // 2, 2), jnp.int32),
            pltpu.VMEM((2, 2, _N), jnp.float32),
            pltpu.VMEM((2 * _GRP,), jnp.float32),
            pltpu.VMEM((16, _D), jnp.float32),
            pltpu.SemaphoreType.DMA((2,)),
            pltpu.SemaphoreType.DMA((2,)),
            pltpu.SemaphoreType.DMA,
        ],
    )
    def k(A_hbm, H_hbm, idx_hbm, idxp_hbm, pA_hbm, pH_hbm,
          cidx, ridx1, ridx2, rowbuf, outbuf, hvbuf, sem_row, sem_out, sem_h):
        wid = lax.axis_index("s") * _NC + lax.axis_index("c")
        start = pl.multiple_of(wid * _CHUNK, _CHUNK)
        half = pl.multiple_of(wid * (_CHUNK // 2), _CHUNK // 2)
        nrows = jnp.where(wid < _NW - 1, _CHUNK, _TAIL)
        nt = nrows // 2
        ng = nrows // 8

        pltpu.sync_copy(idx_hbm, cidx)
        pltpu.sync_copy(idx_hbm.at[pl.ds(start, _CHUNK)], ridx1)
        pltpu.sync_copy(idxp_hbm.at[pl.ds(half, _CHUNK // 2)], ridx2)

        for p in range(_CHUNK // 16):
            pltpu.async_copy(H_hbm.at[ridx1.at[pl.ds(16 * p, 16)]],
                             hvbuf, sem_h).wait()
            dst0 = pl.multiple_of(start + 16 * p, 8)
            pltpu.sync_copy(hvbuf, pH_hbm.at[pl.ds(dst0, 16)])

        def row_gather(t, tb):
            return pltpu.make_async_copy(A_hbm.at[ridx2.at[t]],
                                         rowbuf.at[tb], sem_row.at[tb])

        def grp_store(g, gb):
            src = pl.multiple_of(gb * _GRP, 8)
            dst = pl.multiple_of((start // 8 + g) * _GRP, 8)
            return pltpu.make_async_copy(outbuf.at[pl.ds(src, _GRP)],
                                         pA_hbm.at[pl.ds(dst, _GRP)],
                                         sem_out.at[gb])

        row_gather(0, 0).start()

        def body(t, carry):
            tb = lax.rem(t, 2)
            g = t // 4
            gb = lax.rem(g, 2)
            s = lax.rem(t, 4)

            @pl.when((s == 0) & (g >= 2))
            def _():
                grp_store(g - 2, gb).wait()

            row_gather(t, tb).wait()

            @pl.when(t + 1 < nt)
            def _():
                row_gather(t + 1, 1 - tb).start()

            base0 = pl.multiple_of(gb * _GRP + (2 * s) * 128, 16)
            base1 = pl.multiple_of(gb * _GRP + (2 * s + 1) * 128, 16)
            tbv = jnp.full((16,), tb, jnp.int32)
            r0v = jnp.zeros((16,), jnp.int32)
            r1v = jnp.ones((16,), jnp.int32)

            def cg(jv, c):
                off = (jv >> 3) * 1024 + (jv & 7) * 16
                cvec = cidx[pl.ds(jv * 16, 16)]
                outbuf[pl.ds(base0 + off, 16)] = plsc.load_gather(
                    rowbuf, [tbv, r0v, cvec])
                outbuf[pl.ds(base1 + off, 16)] = plsc.load_gather(
                    rowbuf, [tbv, r1v, cvec])
                return c

            lax.fori_loop(0, _NJ, cg, 0, unroll=4)

            @pl.when(s == 3)
            def _():
                grp_store(g, gb).start()

            return carry

        lax.fori_loop(0, nt, body, 0)
        grp_store(ng - 2, lax.rem(ng - 2, 2)).wait()
        grp_store(ng - 1, lax.rem(ng - 1, 2)).wait()

    return k(A, Hm, idx_pad, idx_pairs)


def _b1(pH, vals, W):
    def body(ph_ref, v_ref, w_ref, o_ref):
        o_ref[...] = jnp.dot(ph_ref[...] * v_ref[...], w_ref[...],
                             preferred_element_type=jnp.float32)

    return pl.pallas_call(
        body,
        out_shape=jax.ShapeDtypeStruct((_K, _D), jnp.float32),
    )(pH, vals, W)


def _b2(tpa, HW2):
    B = 25

    def body(a_ref, hw_ref, o_ref, pa_ref):
        acc = jnp.zeros((B * 8, _D), jnp.float32)
        for j in range(_NT):
            blk = a_ref[:, j].reshape(B * 8, 128)
            acc = acc + jnp.dot(blk, hw_ref[j],
                                preferred_element_type=jnp.float32)
            c0 = 128 * j
            if c0 + 128 <= _K:
                pa_ref[:, c0:c0 + 128] = blk
            else:
                pa_ref[:, c0:_K] = blk[:, :_K - c0]
        o_ref[...] = jnp.maximum(acc, 0.0)

    return pl.pallas_call(
        body,
        grid=(_K // (B * 8),),
        in_specs=[pl.BlockSpec((B, _NT, 8, 128), lambda i: (i, 0, 0, 0)),
                  pl.BlockSpec((_NT, 128, 128), lambda i: (0, 0, 0))],
        out_specs=(pl.BlockSpec((B * 8, _D), lambda i: (i, 0)),
                   pl.BlockSpec((B * 8, _K), lambda i: (i, 0))),
        out_shape=(jax.ShapeDtypeStruct((_K, _D), jnp.float32),
                   jax.ShapeDtypeStruct((_K, _K), jnp.float32)),
    )(tpa, HW2)


def kernel(H, A, W, proj_W, proj_b):
    weights = (H @ proj_W + proj_b)[:, 0]
    scores = jax.nn.sigmoid(weights)

    spad = jnp.concatenate(
        [scores, jnp.full((_NPAD - _N,), -jnp.inf, jnp.float32)])
    ks, ki = _topk_sort(spad.reshape(_SORT_R, _SORT_C))
    values = ks.reshape(-1)[:_K]
    idx = ki.reshape(-1)[:_K]

    idx_pad = jnp.concatenate([idx, jnp.zeros((_RPAD - _K,), jnp.int32)])
    tpaf, pHp = _sc_pool(A, H, idx_pad, idx_pad.reshape(_RPAD // 2, 2))
    tpa = tpaf.reshape(_K // 8, _NT, 8, 128)
    pH = pHp[:_K]

    HW = _b1(pH, values.reshape(_K, 1), W)
    HW2 = jnp.concatenate(
        [HW, jnp.zeros((_KPAD - _K, _D), jnp.float32)]).reshape(_NT, 128, _D)
    out, pA = _b2(tpa, HW2)
    return (out, pA, idx)


# speedup vs baseline: 2.2753x; 1.0013x over previous
import functools

import jax
import jax.numpy as jnp
from jax import lax
from jax.experimental import pallas as pl
from jax.experimental.pallas import tpu as pltpu
from jax.experimental.pallas import tpu_sc as plsc

_N = 10000
_D = 128
_K = 5000
_NPAD = 16384
_SORT_R = 128
_SORT_C = 128
_KPAD = 5120
_NJ = _KPAD // 16
_NT = _KPAD // 128
_GRP = 8 * _KPAD
_RPAD = 5120

_NC = 2
_NS = 16
_NW = _NC * _NS
_CHUNK = 160
_TAIL = _K - (_NW - 1) * _CHUNK


def _roll(x, shift, axis):
    n = x.shape[axis]
    shift = shift % n
    if shift == 0:
        return x
    if axis == 0:
        return jnp.concatenate([x[shift:, :], x[:shift, :]], axis=0)
    return jnp.concatenate([x[:, shift:], x[:, :shift]], axis=1)


def _sort_body(s_ref, ks_ref, ki_ref):
    S = s_ref[...]
    row = lax.broadcasted_iota(jnp.int32, (_SORT_R, _SORT_C), 0)
    col = lax.broadcasted_iota(jnp.int32, (_SORT_R, _SORT_C), 1)
    L = row * _SORT_C + col
    I = L
    k = 2
    while k <= _NPAD:
        d = k // 2
        while d >= 1:
            if d < _SORT_C:
                ax, sh = 1, d
            else:
                ax, sh = 0, d // _SORT_C
            lower = (L & d) == 0
            oS = jnp.where(lower, _roll(S, sh, ax), _roll(S, -sh, ax))
            oI = jnp.where(lower, _roll(I, sh, ax), _roll(I, -sh, ax))
            lt = (S > oS) | ((S == oS) & (I < oI))
            asc = (L & k) == 0
            take = lt == (lower == asc)
            S = jnp.where(take, S, oS)
            I = jnp.where(take, I, oI)
            d //= 2
        k *= 2
    ks_ref[...] = S
    ki_ref[...] = I


def _topk_sort(spad):
    return pl.pallas_call(
        _sort_body,
        out_shape=(jax.ShapeDtypeStruct((_SORT_R, _SORT_C), jnp.float32),
                   jax.ShapeDtypeStruct((_SORT_R, _SORT_C), jnp.int32)),
    )(spad)


def _sc_pool(A, Hm, idx_pad, idx_pairs):
    mesh = plsc.VectorSubcoreMesh(core_axis_name="c", subcore_axis_name="s")

    @functools.partial(
        pl.kernel,
        out_type=(jax.ShapeDtypeStruct((_K // 8, _NT, 8, 128), jnp.float32),
                  jax.ShapeDtypeStruct((_RPAD, _D), jnp.float32)),
        mesh=mesh,
        compiler_params=pltpu.CompilerParams(needs_layout_passes=False,
                                             use_tc_tiling_on_sc=False),
        scratch_types=[
            pltpu.VMEM((_KPAD,), jnp.int32),
            pltpu.VMEM((_CHUNK,), jnp.int32),
            pltpu.VMEM((_CHUNK // 2, 2), jnp.int32),
            pltpu.VMEM((2, 2, _N), jnp.float32),
            pltpu.VMEM((2, _NT, 8, 128), jnp.float32),
            pltpu.VMEM((16, _D), jnp.float32),
            pltpu.SemaphoreType.DMA((2,)),
            pltpu.SemaphoreType.DMA((2,)),
            pltpu.SemaphoreType.DMA,
        ],
    )
    def k(A_hbm, H_hbm, idx_hbm, idxp_hbm, pA_hbm, pH_hbm,
          cidx, ridx1, ridx2, rowbuf, outbuf, hvbuf, sem_row, sem_out, sem_h):
        wid = lax.axis_index("s") * _NC + lax.axis_index("c")
        start = pl.multiple_of(wid * _CHUNK, _CHUNK)
        half = pl.multiple_of(wid * (_CHUNK // 2), _CHUNK // 2)
        nrows = jnp.where(wid < _NW - 1, _CHUNK, _TAIL)
        nt = nrows // 2
        ng = nrows // 8

        pltpu.sync_copy(idx_hbm, cidx)
        pltpu.sync_copy(idx_hbm.at[pl.ds(start, _CHUNK)], ridx1)
        pltpu.sync_copy(idxp_hbm.at[pl.ds(half, _CHUNK // 2)], ridx2)

        for p in range(_CHUNK // 16):
            pltpu.async_copy(H_hbm.at[ridx1.at[pl.ds(16 * p, 16)]],
                             hvbuf, sem_h).wait()
            dst0 = pl.multiple_of(start + 16 * p, 8)
            pltpu.sync_copy(hvbuf, pH_hbm.at[pl.ds(dst0, 16)])

        def row_gather(t, tb):
            return pltpu.make_async_copy(A_hbm.at[ridx2.at[t]],
                                         rowbuf.at[tb], sem_row.at[tb])

        def grp_store(g, gb):
            return pltpu.make_async_copy(outbuf.at[gb],
                                         pA_hbm.at[start // 8 + g],
                                         sem_out.at[gb])

        row_gather(0, 0).start()

        def body(t, carry):
            tb = lax.rem(t, 2)
            g = t // 4
            gb = lax.rem(g, 2)
            s = lax.rem(t, 4)

            @pl.when((s == 0) & (g >= 2))
            def _():
                grp_store(g - 2, gb).wait()

            row_gather(t, tb).wait()

            @pl.when(t + 1 < nt)
            def _():
                row_gather(t + 1, 1 - tb).start()

            tbv = jnp.full((16,), tb, jnp.int32)
            r0v = jnp.zeros((16,), jnp.int32)
            r1v = jnp.ones((16,), jnp.int32)

            def cg(jv, c):
                tj = jv >> 3
                off = (jv & 7) * 16
                cvec = cidx[pl.ds(jv * 16, 16)]
                outbuf[gb, tj, 2 * s, pl.ds(off, 16)] = plsc.load_gather(
                    rowbuf, [tbv, r0v, cvec])
                outbuf[gb, tj, 2 * s + 1, pl.ds(off, 16)] = plsc.load_gather(
                    rowbuf, [tbv, r1v, cvec])
                return c

            lax.fori_loop(0, _NJ, cg, 0, unroll=4)

            @pl.when(s == 3)
            def _():
                grp_store(g, gb).start()

            return carry

        lax.fori_loop(0, nt, body, 0)
        grp_store(ng - 2, lax.rem(ng - 2, 2)).wait()
        grp_store(ng - 1, lax.rem(ng - 1, 2)).wait()

    return k(A, Hm, idx_pad, idx_pairs)


def _b1(pH, vals, W):
    def body(ph_ref, v_ref, w_ref, o_ref):
        o_ref[...] = jnp.dot(ph_ref[...] * v_ref[...], w_ref[...],
                             preferred_element_type=jnp.float32)

    return pl.pallas_call(
        body,
        out_shape=jax.ShapeDtypeStruct((_K, _D), jnp.float32),
    )(pH, vals, W)


def _b2(tpa, HW2):
    B = 25

    def body(a_ref, hw_ref, o_ref, pa_ref):
        acc = jnp.zeros((B * 8, _D), jnp.float32)
        for j in range(_NT):
            blk = a_ref[:, j].reshape(B * 8, 128)
            acc = acc + jnp.dot(blk, hw_ref[j],
                                preferred_element_type=jnp.float32)
            c0 = 128 * j
            if c0 + 128 <= _K:
                pa_ref[:, c0:c0 + 128] = blk
            else:
                pa_ref[:, c0:_K] = blk[:, :_K - c0]
        o_ref[...] = jnp.maximum(acc, 0.0)

    return pl.pallas_call(
        body,
        grid=(_K // (B * 8),),
        in_specs=[pl.BlockSpec((B, _NT, 8, 128), lambda i: (i, 0, 0, 0)),
                  pl.BlockSpec((_NT, 128, 128), lambda i: (0, 0, 0))],
        out_specs=(pl.BlockSpec((B * 8, _D), lambda i: (i, 0)),
                   pl.BlockSpec((B * 8, _K), lambda i: (i, 0))),
        out_shape=(jax.ShapeDtypeStruct((_K, _D), jnp.float32),
                   jax.ShapeDtypeStruct((_K, _K), jnp.float32)),
    )(tpa, HW2)


def kernel(H, A, W, proj_W, proj_b):
    weights = (H @ proj_W + proj_b)[:, 0]
    scores = jax.nn.sigmoid(weights)

    spad = jnp.concatenate(
        [scores, jnp.full((_NPAD - _N,), -jnp.inf, jnp.float32)])
    ks, ki = _topk_sort(spad.reshape(_SORT_R, _SORT_C))
    values = ks.reshape(-1)[:_K]
    idx = ki.reshape(-1)[:_K]

    idx_pad = jnp.concatenate([idx, jnp.zeros((_RPAD - _K,), jnp.int32)])
    tpa, pHp = _sc_pool(A, H, idx_pad, idx_pad.reshape(_RPAD // 2, 2))
    pH = pHp[:_K]

    HW = _b1(pH, values.reshape(_K, 1), W)
    HW2 = jnp.concatenate(
        [HW, jnp.zeros((_KPAD - _K, _D), jnp.float32)]).reshape(_NT, 128, _D)
    out, pA = _b2(tpa, HW2)
    return (out, pA, idx)


# speedup vs baseline: 2.2756x; 1.0001x over previous
import functools

import jax
import jax.numpy as jnp
from jax import lax
from jax.experimental import pallas as pl
from jax.experimental.pallas import tpu as pltpu
from jax.experimental.pallas import tpu_sc as plsc

_N = 10000
_D = 128
_K = 5000
_NPAD = 16384
_SORT_R = 128
_SORT_C = 128
_KPAD = 5120
_NJ = _KPAD // 16
_NT = _KPAD // 128
_GRP = 8 * _KPAD
_RPAD = 5120

_NC = 2
_NS = 16
_NW = _NC * _NS
_CHUNK = 160
_TAIL = _K - (_NW - 1) * _CHUNK


def _roll(x, shift, axis):
    n = x.shape[axis]
    shift = shift % n
    if shift == 0:
        return x
    if axis == 0:
        return jnp.concatenate([x[shift:, :], x[:shift, :]], axis=0)
    return jnp.concatenate([x[:, shift:], x[:, :shift]], axis=1)


def _sort_body(s_ref, ks_ref, ki_ref):
    S = s_ref[...]
    row = lax.broadcasted_iota(jnp.int32, (_SORT_R, _SORT_C), 0)
    col = lax.broadcasted_iota(jnp.int32, (_SORT_R, _SORT_C), 1)
    L = row * _SORT_C + col
    I = L
    k = 2
    while k <= _NPAD:
        d = k // 2
        while d >= 1:
            if d < _SORT_C:
                ax, sh = 1, d
            else:
                ax, sh = 0, d // _SORT_C
            lower = (L & d) == 0
            oS = jnp.where(lower, _roll(S, sh, ax), _roll(S, -sh, ax))
            oI = jnp.where(lower, _roll(I, sh, ax), _roll(I, -sh, ax))
            lt = (S > oS) | ((S == oS) & (I < oI))
            asc = (L & k) == 0
            take = lt == (lower == asc)
            S = jnp.where(take, S, oS)
            I = jnp.where(take, I, oI)
            d //= 2
        k *= 2
    ks_ref[...] = S
    ki_ref[...] = I


def _topk_sort(spad):
    return pl.pallas_call(
        _sort_body,
        out_shape=(jax.ShapeDtypeStruct((_SORT_R, _SORT_C), jnp.float32),
                   jax.ShapeDtypeStruct((_SORT_R, _SORT_C), jnp.int32)),
    )(spad)


def _sc_pool(A, Hm, idx_pad, idx_pairs):
    mesh = plsc.VectorSubcoreMesh(core_axis_name="c", subcore_axis_name="s")

    @functools.partial(
        pl.kernel,
        out_type=(jax.ShapeDtypeStruct((_K // 8 * _NT * 8, 128), jnp.float32),
                  jax.ShapeDtypeStruct((_RPAD, _D), jnp.float32)),
        mesh=mesh,
        compiler_params=pltpu.CompilerParams(needs_layout_passes=False,
                                             use_tc_tiling_on_sc=False),
        scratch_types=[
            pltpu.VMEM((_KPAD,), jnp.int32),
            pltpu.VMEM((_CHUNK,), jnp.int32),
            pltpu.VMEM((_CHUNK // 2, 2), jnp.int32),
            pltpu.VMEM((2, 2, _N), jnp.float32),
            pltpu.VMEM((2, _NT * 8, 128), jnp.float32),
            pltpu.VMEM((16, _D), jnp.float32),
            pltpu.SemaphoreType.DMA((2,)),
            pltpu.SemaphoreType.DMA((2,)),
            pltpu.SemaphoreType.DMA,
        ],
    )
    def k(A_hbm, H_hbm, idx_hbm, idxp_hbm, pA_hbm, pH_hbm,
          cidx, ridx1, ridx2, rowbuf, outbuf, hvbuf, sem_row, sem_out, sem_h):
        wid = lax.axis_index("s") * _NC + lax.axis_index("c")
        start = pl.multiple_of(wid * _CHUNK, _CHUNK)
        half = pl.multiple_of(wid * (_CHUNK // 2), _CHUNK // 2)
        nrows = jnp.where(wid < _NW - 1, _CHUNK, _TAIL)
        nt = nrows // 2
        ng = nrows // 8

        pltpu.sync_copy(idx_hbm, cidx)
        pltpu.sync_copy(idx_hbm.at[pl.ds(start, _CHUNK)], ridx1)
        pltpu.sync_copy(idxp_hbm.at[pl.ds(half, _CHUNK // 2)], ridx2)

        for p in range(_CHUNK // 16):
            pltpu.async_copy(H_hbm.at[ridx1.at[pl.ds(16 * p, 16)]],
                             hvbuf, sem_h).wait()
            dst0 = pl.multiple_of(start + 16 * p, 8)
            pltpu.sync_copy(hvbuf, pH_hbm.at[pl.ds(dst0, 16)])

        def row_gather(t, tb):
            return pltpu.make_async_copy(A_hbm.at[ridx2.at[t]],
                                         rowbuf.at[tb], sem_row.at[tb])

        def grp_store(g, gb):
            dst = pl.multiple_of((start // 8 + g) * (_NT * 8), 8)
            return pltpu.make_async_copy(outbuf.at[gb],
                                         pA_hbm.at[pl.ds(dst, _NT * 8)],
                                         sem_out.at[gb])

        row_gather(0, 0).start()

        def body(t, carry):
            tb = lax.rem(t, 2)
            g = t // 4
            gb = lax.rem(g, 2)
            s = lax.rem(t, 4)

            @pl.when((s == 0) & (g >= 2))
            def _():
                grp_store(g - 2, gb).wait()

            row_gather(t, tb).wait()

            @pl.when(t + 1 < nt)
            def _():
                row_gather(t + 1, 1 - tb).start()

            tbv = jnp.full((16,), tb, jnp.int32)
            r0v = jnp.zeros((16,), jnp.int32)
            r1v = jnp.ones((16,), jnp.int32)

            def cg(jv, c):
                tj = jv >> 3
                off = (jv & 7) * 16
                cvec = cidx[pl.ds(jv * 16, 16)]
                outbuf[gb, tj * 8 + 2 * s, pl.ds(off, 16)] = plsc.load_gather(
                    rowbuf, [tbv, r0v, cvec])
                outbuf[gb, tj * 8 + 2 * s + 1, pl.ds(off, 16)] = (
                    plsc.load_gather(rowbuf, [tbv, r1v, cvec]))
                return c

            lax.fori_loop(0, _NJ, cg, 0, unroll=4)

            @pl.when(s == 3)
            def _():
                grp_store(g, gb).start()

            return carry

        lax.fori_loop(0, nt, body, 0)
        grp_store(ng - 2, lax.rem(ng - 2, 2)).wait()
        grp_store(ng - 1, lax.rem(ng - 1, 2)).wait()

    return k(A, Hm, idx_pad, idx_pairs)


def _b1(pH, vals, W):
    def body(ph_ref, v_ref, w_ref, o_ref):
        o_ref[...] = jnp.dot(ph_ref[...] * v_ref[...], w_ref[...],
                             preferred_element_type=jnp.float32)

    return pl.pallas_call(
        body,
        out_shape=jax.ShapeDtypeStruct((_K, _D), jnp.float32),
    )(pH, vals, W)


def _b2(tpa, HW2):
    B = 25
    BR = B * _NT * 8
    G = _K // (B * 8)

    def body(tpa_hbm, hw_ref, o_ref, pa_ref, buf, sem):
        i = pl.program_id(0)

        def fetch(step, sb):
            return pltpu.make_async_copy(
                tpa_hbm.at[pl.ds(step * BR, BR)], buf.at[sb], sem.at[sb])

        @pl.when(i == 0)
        def _():
            fetch(0, 0).start()

        @pl.when(i + 1 < G)
        def _():
            fetch(i + 1, (i + 1) % 2).start()

        b = i % 2
        fetch(i, b).wait()
        v = buf[b].reshape(B, _NT, 8, 128)
        acc = jnp.zeros((B * 8, _D), jnp.float32)
        for j in range(_NT):
            blk = v[:, j].reshape(B * 8, 128)
            acc = acc + jnp.dot(blk, hw_ref[j],
                                preferred_element_type=jnp.float32)
            c0 = 128 * j
            if c0 + 128 <= _K:
                pa_ref[:, c0:c0 + 128] = blk
            else:
                pa_ref[:, c0:_K] = blk[:, :_K - c0]
        o_ref[...] = jnp.maximum(acc, 0.0)

    return pl.pallas_call(
        body,
        grid=(G,),
        in_specs=[pl.BlockSpec(memory_space=pltpu.MemorySpace.HBM),
                  pl.BlockSpec((_NT, 128, 128), lambda i: (0, 0, 0))],
        out_specs=(pl.BlockSpec((B * 8, _D), lambda i: (i, 0)),
                   pl.BlockSpec((B * 8, _K), lambda i: (i, 0))),
        out_shape=(jax.ShapeDtypeStruct((_K, _D), jnp.float32),
                   jax.ShapeDtypeStruct((_K, _K), jnp.float32)),
        scratch_shapes=[pltpu.VMEM((2, BR, 128), jnp.float32),
                        pltpu.SemaphoreType.DMA((2,))],
    )(tpa, HW2)


def kernel(H, A, W, proj_W, proj_b):
    weights = (H @ proj_W + proj_b)[:, 0]
    scores = jax.nn.sigmoid(weights)

    spad = jnp.concatenate(
        [scores, jnp.full((_NPAD - _N,), -jnp.inf, jnp.float32)])
    ks, ki = _topk_sort(spad.reshape(_SORT_R, _SORT_C))
    values = ks.reshape(-1)[:_K]
    idx = ki.reshape(-1)[:_K]

    idx_pad = jnp.concatenate([idx, jnp.zeros((_RPAD - _K,), jnp.int32)])
    tpaf, pHp = _sc_pool(A, H, idx_pad, idx_pad.reshape(_RPAD // 2, 2))
    pH = pHp[:_K]

    HW = _b1(pH, values.reshape(_K, 1), W)
    HW2 = jnp.concatenate(
        [HW, jnp.zeros((_KPAD - _K, _D), jnp.float32)]).reshape(_NT, 128, _D)
    out, pA = _b2(tpaf, HW2)
    return (out, pA, idx)


# speedup vs baseline: 3.4390x; 1.5113x over previous
import functools

import jax
import jax.numpy as jnp
from jax import lax
from jax.experimental import pallas as pl
from jax.experimental.pallas import tpu as pltpu
from jax.experimental.pallas import tpu_sc as plsc

_N = 10000
_D = 128
_K = 5000
_NPAD = 16384
_SORT_R = 128
_SORT_C = 128
_KPAD = 5120
_NJ = _KPAD // 16
_NT = _KPAD // 128
_GRP = 8 * _KPAD
_RPAD = 5120
_NMAIN = 9984
_TSTART = _N - 128
_NB = _NMAIN + 128

_NC = 2
_NS = 16
_NW = _NC * _NS
_CHUNK = 160
_TAIL = _K - (_NW - 1) * _CHUNK


def _roll(x, shift, axis):
    n = x.shape[axis]
    shift = shift % n
    if shift == 0:
        return x
    if axis == 0:
        return jnp.concatenate([x[shift:, :], x[:shift, :]], axis=0)
    return jnp.concatenate([x[:, shift:], x[:, :shift]], axis=1)


def _sort_body(s_ref, ks_ref, ki_ref):
    S = s_ref[...]
    row = lax.broadcasted_iota(jnp.int32, (_SORT_R, _SORT_C), 0)
    col = lax.broadcasted_iota(jnp.int32, (_SORT_R, _SORT_C), 1)
    L = row * _SORT_C + col
    I = L
    k = 2
    while k <= _NPAD:
        d = k // 2
        while d >= 1:
            if d < _SORT_C:
                ax, sh = 1, d
            else:
                ax, sh = 0, d // _SORT_C
            lower = (L & d) == 0
            oS = jnp.where(lower, _roll(S, sh, ax), _roll(S, -sh, ax))
            oI = jnp.where(lower, _roll(I, sh, ax), _roll(I, -sh, ax))
            lt = (S > oS) | ((S == oS) & (I < oI))
            asc = (L & k) == 0
            take = lt == (lower == asc)
            S = jnp.where(take, S, oS)
            I = jnp.where(take, I, oI)
            d //= 2
        k *= 2
    ks_ref[...] = S
    ki_ref[...] = I


def _topk_sort(spad):
    return pl.pallas_call(
        _sort_body,
        out_shape=(jax.ShapeDtypeStruct((_SORT_R, _SORT_C), jnp.float32),
                   jax.ShapeDtypeStruct((_SORT_R, _SORT_C), jnp.int32)),
    )(spad)


def _sc_pool(A, Atail, Hm, idx_pad, idx_pairs):
    mesh = plsc.VectorSubcoreMesh(core_axis_name="c", subcore_axis_name="s")

    @functools.partial(
        pl.kernel,
        out_type=(jax.ShapeDtypeStruct((_K // 8 * _NT * 8, 128), jnp.float32),
                  jax.ShapeDtypeStruct((_RPAD, _D), jnp.float32)),
        mesh=mesh,
        compiler_params=pltpu.CompilerParams(needs_layout_passes=False,
                                             use_tc_tiling_on_sc=True),
        scratch_types=[
            pltpu.VMEM((_KPAD,), jnp.int32),
            pltpu.VMEM((_CHUNK,), jnp.int32),
            pltpu.VMEM((_CHUNK // 2, 2), jnp.int32),
            pltpu.VMEM((2, 2, _NB), jnp.float32),
            pltpu.VMEM((_NT * 8, 128), jnp.float32),
            pltpu.VMEM((16, _D), jnp.float32),
            pltpu.SemaphoreType.DMA((2,)),
            pltpu.SemaphoreType.DMA((2,)),
            pltpu.SemaphoreType.DMA((2,)),
            pltpu.SemaphoreType.DMA,
        ],
    )
    def k(A_hbm, At_hbm, H_hbm, idx_hbm, idxp_hbm, pA_hbm, pH_hbm,
          cidx, ridx1, ridx2, rowbuf, outbuf, hvbuf, sem_row, sem_tail,
          sem_out, sem_h):
        wid = lax.axis_index("s") * _NC + lax.axis_index("c")
        start = pl.multiple_of(wid * _CHUNK, _CHUNK)
        half = pl.multiple_of(wid * (_CHUNK // 2), _CHUNK // 2)
        nrows = jnp.where(wid < _NW - 1, _CHUNK, _TAIL)
        nt = nrows // 2
        ng = nrows // 8

        pltpu.sync_copy(idx_hbm, cidx)

        def remap(j, c):
            v = cidx[pl.ds(j * 16, 16)]
            cidx[pl.ds(j * 16, 16)] = jnp.where(v >= _NMAIN, v + 112, v)
            return c

        lax.fori_loop(0, _NJ, remap, 0, unroll=8)
        pltpu.sync_copy(idx_hbm.at[pl.ds(start, _CHUNK)], ridx1)
        pltpu.sync_copy(idxp_hbm.at[pl.ds(half, _CHUNK // 2)], ridx2)

        for p in range(_CHUNK // 16):
            pltpu.async_copy(H_hbm.at[ridx1.at[pl.ds(16 * p, 16)]],
                             hvbuf, sem_h).wait()
            dst0 = pl.multiple_of(start + 16 * p, 8)
            pltpu.sync_copy(hvbuf, pH_hbm.at[pl.ds(dst0, 16)])

        def row_gather(t, tb):
            return pltpu.make_async_copy(
                A_hbm.at[ridx2.at[t], pl.ds(0, _NMAIN)],
                rowbuf.at[tb, :, pl.ds(0, _NMAIN)], sem_row.at[tb])

        def tail_gather(t, tb):
            return pltpu.make_async_copy(
                At_hbm.at[ridx2.at[t]],
                rowbuf.at[tb, :, pl.ds(_NMAIN, 128)], sem_tail.at[tb])

        def grp_store(g):
            dst = pl.multiple_of((start // 8 + g) * (_NT * 8), 8)
            return pltpu.make_async_copy(outbuf,
                                         pA_hbm.at[pl.ds(dst, _NT * 8)],
                                         sem_out.at[0])

        row_gather(0, 0).start()
        tail_gather(0, 0).start()

        def body(t, carry):
            tb = lax.rem(t, 2)
            g = t // 4
            s = lax.rem(t, 4)

            @pl.when((s == 0) & (g >= 1))
            def _():
                grp_store(g - 1).wait()

            row_gather(t, tb).wait()
            tail_gather(t, tb).wait()

            @pl.when(t + 1 < nt)
            def _():
                row_gather(t + 1, 1 - tb).start()
                tail_gather(t + 1, 1 - tb).start()

            tbv = jnp.full((16,), tb, jnp.int32)
            r0v = jnp.zeros((16,), jnp.int32)
            r1v = jnp.ones((16,), jnp.int32)

            def cg(jv, c):
                tj = jv >> 3
                off = (jv & 7) * 16
                cvec = cidx[pl.ds(jv * 16, 16)]
                outbuf[tj * 8 + 2 * s, pl.ds(off, 16)] = plsc.load_gather(
                    rowbuf, [tbv, r0v, cvec])
                outbuf[tj * 8 + 2 * s + 1, pl.ds(off, 16)] = (
                    plsc.load_gather(rowbuf, [tbv, r1v, cvec]))
                return c

            lax.fori_loop(0, _NJ, cg, 0, unroll=4)

            @pl.when(s == 3)
            def _():
                grp_store(g).start()

            return carry

        lax.fori_loop(0, nt, body, 0)
        grp_store(ng - 1).wait()

    return k(A, Atail, Hm, idx_pad, idx_pairs)


def _b1(pH, vals, W):
    def body(ph_ref, v_ref, w_ref, o_ref):
        o_ref[...] = jnp.dot(ph_ref[...] * v_ref[...], w_ref[...],
                             preferred_element_type=jnp.float32)

    return pl.pallas_call(
        body,
        out_shape=jax.ShapeDtypeStruct((_K, _D), jnp.float32),
    )(pH, vals, W)


def _b2(tpa, HW2):
    B = 25
    BR = B * _NT * 8
    G = _K // (B * 8)

    def body(tpa_hbm, hw_ref, o_ref, pa_ref, buf, sem):
        i = pl.program_id(0)

        def fetch(step, sb):
            return pltpu.make_async_copy(
                tpa_hbm.at[pl.ds(step * BR, BR)], buf.at[sb], sem.at[sb])

        @pl.when(i == 0)
        def _():
            fetch(0, 0).start()

        @pl.when(i + 1 < G)
        def _():
            fetch(i + 1, (i + 1) % 2).start()

        b = i % 2
        fetch(i, b).wait()
        v = buf[b].reshape(B, _NT, 8, 128)
        acc = jnp.zeros((B * 8, _D), jnp.float32)
        for j in range(_NT):
            blk = v[:, j].reshape(B * 8, 128)
            acc = acc + jnp.dot(blk, hw_ref[j],
                                preferred_element_type=jnp.float32)
            c0 = 128 * j
            if c0 + 128 <= _K:
                pa_ref[:, c0:c0 + 128] = blk
            else:
                pa_ref[:, c0:_K] = blk[:, :_K - c0]
        o_ref[...] = jnp.maximum(acc, 0.0)

    return pl.pallas_call(
        body,
        grid=(G,),
        in_specs=[pl.BlockSpec(memory_space=pltpu.MemorySpace.HBM),
                  pl.BlockSpec((_NT, 128, 128), lambda i: (0, 0, 0))],
        out_specs=(pl.BlockSpec((B * 8, _D), lambda i: (i, 0)),
                   pl.BlockSpec((B * 8, _K), lambda i: (i, 0))),
        out_shape=(jax.ShapeDtypeStruct((_K, _D), jnp.float32),
                   jax.ShapeDtypeStruct((_K, _K), jnp.float32)),
        scratch_shapes=[pltpu.VMEM((2, BR, 128), jnp.float32),
                        pltpu.SemaphoreType.DMA((2,))],
    )(tpa, HW2)


def kernel(H, A, W, proj_W, proj_b):
    weights = (H @ proj_W + proj_b)[:, 0]
    scores = jax.nn.sigmoid(weights)

    spad = jnp.concatenate(
        [scores, jnp.full((_NPAD - _N,), -jnp.inf, jnp.float32)])
    ks, ki = _topk_sort(spad.reshape(_SORT_R, _SORT_C))
    values = ks.reshape(-1)[:_K]
    idx = ki.reshape(-1)[:_K]

    idx_pad = jnp.concatenate([idx, jnp.zeros((_RPAD - _K,), jnp.int32)])
    Atail = lax.slice(A, (0, _TSTART), (_N, _N))
    tpaf, pHp = _sc_pool(A, Atail, H, idx_pad,
                         idx_pad.reshape(_RPAD // 2, 2))
    pH = pHp[:_K]

    HW = _b1(pH, values.reshape(_K, 1), W)
    HW2 = jnp.concatenate(
        [HW, jnp.zeros((_KPAD - _K, _D), jnp.float32)]).reshape(_NT, 128, _D)
    out, pA = _b2(tpaf, HW2)
    return (out, pA, idx)
